# baseline (speedup 1.0000x reference)
import functools

import jax
import jax.numpy as jnp
from jax import lax
from jax.experimental import pallas as pl
from jax.experimental.pallas import tpu as pltpu
from jax.experimental.pallas import tpu_sc as plsc

B = 128
V = 100000
B_SC = 32
B_TC = B - B_SC
W = 2048
C = (V + W - 1) // W

KS0 = 0
KS1 = 42
KS2 = KS0 ^ KS1 ^ 0x1BD11BDA
_ROTS = ((13, 15, 26, 6), (17, 29, 16, 24))
_INJECT = ((KS1, KS2, 1), (KS2, KS0, 2), (KS0, KS1, 3), (KS1, KS2, 4), (KS2, KS0, 5))

_TINY = float(jnp.finfo(jnp.float32).tiny)
_NEG_INF = float("-inf")

_LOG_POLY = (
    -3.173079160534442e-11,
    1.0000000025276106,
    -0.4999999820678256,
    0.33333278012578005,
    -0.25000127717187504,
    0.20003420797455176,
    -0.16665529578122412,
    0.14199694268775429,
    -0.12424601284408243,
    0.12017414115663498,
    -0.11631797397291235,
    0.06459239173209066,
)
_LN2 = 0.6931471805599453
_SQRT2 = 1.4142135623730951


def _threefry_bits(cnt, u32):
    if u32:
        def rotl(x, r):
            return (x << jnp.uint32(r)) | (x >> jnp.uint32(32 - r))
        cst = jnp.uint32
    else:
        def rotl(x, r):
            return (x << jnp.int32(r)) | lax.shift_right_logical(x, jnp.int32(32 - r))
        def cst(v):
            v &= 0xFFFFFFFF
            return jnp.int32(v - 0x100000000 if v >= 0x80000000 else v)
    x0 = cnt + cst(KS1)
    x1 = rotl(x0, 13) ^ x0
    first = True
    for g in range(5):
        for r in _ROTS[g % 2]:
            if first:
                first = False
                continue
            x0 = x0 + x1
            x1 = rotl(x1, r) ^ x0
        a, b, c = _INJECT[g]
        if a:
            x0 = x0 + cst(a)
        x1 = x1 + cst(b + c)
    return x0 ^ x1


def _tc_body(logits_ref, out_ref, m_ref, i_ref):
    j = pl.program_id(0)

    @pl.when(j == 0)
    def _init():
        m_ref[...] = jnp.full_like(m_ref, jnp.float32(_NEG_INF))
        i_ref[...] = jnp.zeros_like(i_ref)

    x = logits_ref[...]
    row = lax.broadcasted_iota(jnp.int32, (B_TC, W), 0)
    col = lax.broadcasted_iota(jnp.int32, (B_TC, W), 1) + j * W
    cnt = (row * V + col).astype(jnp.uint32)

    bits = _threefry_bits(cnt, u32=True)
    fl = lax.bitcast_convert_type(
        (bits >> jnp.uint32(9)) | jnp.uint32(0x3F800000), jnp.float32
    ) - jnp.float32(1.0)
    u = jnp.maximum(jnp.float32(_TINY), fl)
    g = -jnp.log(-jnp.log(u))
    vals = jnp.where(col < V, x + g, jnp.float32(_NEG_INF))

    bm = jnp.max(vals, axis=1, keepdims=True)
    bi = jnp.min(
        jnp.where(vals == bm, col, jnp.int32(0x7FFFFFFF)), axis=1, keepdims=True
    )

    pm = m_ref[:, 0:1]
    pi = i_ref[:, 0:1]
    better = bm > pm
    nm = jnp.where(better, bm, pm)
    ni = jnp.where(better, bi, pi)
    m_ref[...] = jnp.broadcast_to(nm, m_ref.shape)
    i_ref[...] = jnp.broadcast_to(ni, i_ref.shape)

    @pl.when(j == C - 1)
    def _fin():
        out_ref[...] = jnp.broadcast_to(ni, out_ref.shape)


def _tc_sample(logits_top, interpret=False):
    out = pl.pallas_call(
        _tc_body,
        grid=(C,),
        in_specs=[pl.BlockSpec((B_TC, W), lambda j: (0, j))],
        out_specs=pl.BlockSpec((B_TC, 128), lambda j: (0, 0)),
        out_shape=jax.ShapeDtypeStruct((B_TC, 128), jnp.int32),
        scratch_shapes=[
            pltpu.VMEM((B_TC, 128), jnp.float32),
            pltpu.VMEM((B_TC, 128), jnp.int32),
        ],
        compiler_params=pltpu.CompilerParams(
            dimension_semantics=("arbitrary",),
        ),
        interpret=interpret,
    )(logits_top)
    return out[:, 0]


CH = 2000
NCH = V // CH
NVEC = CH // 16


def _neg_log(u):
    bx = lax.bitcast_convert_type(u, jnp.int32)
    e = lax.shift_right_logical(bx, jnp.int32(23)) - jnp.int32(127)
    m = lax.bitcast_convert_type(
        (bx & jnp.int32(0x007FFFFF)) | jnp.int32(0x3F800000), jnp.float32
    )
    big = m >= jnp.float32(_SQRT2)
    m = jnp.where(big, m * jnp.float32(0.5), m)
    ef = e.astype(jnp.float32)
    ef = jnp.where(big, ef + jnp.float32(1.0), ef)
    t = m - jnp.float32(1.0)
    acc = jnp.float32(_LOG_POLY[-1])
    for c in _LOG_POLY[-2::-1]:
        acc = acc * t + jnp.float32(c)
    return -(ef * jnp.float32(_LN2) + acc)


def _sc_body(logits_hbm, out_hbm, buf, mbuf, ibuf):
    cix = lax.axis_index("c")
    six = lax.axis_index("s")
    w = six * 2 + cix
    rowg = w + B_TC
    base = rowg * V

    def chunk_body(j, carry):
        m, idx = carry
        pltpu.sync_copy(logits_hbm.at[pl.ds(w * V + j * CH, CH)], buf)

        def vec_body(v, carry):
            m, idx = carry
            l = buf[pl.ds(v * 16, 16)]
            col0 = j * CH + v * 16
            colv = lax.iota(jnp.int32, 16) + col0
            cnt = colv + base
            bits = _threefry_bits(cnt, u32=False)
            fl = lax.bitcast_convert_type(
                lax.shift_right_logical(bits, jnp.int32(9))
                | jnp.int32(0x3F800000),
                jnp.float32,
            ) - jnp.float32(1.0)
            u = jnp.maximum(jnp.float32(_TINY), fl)
            key = _neg_log(u) * jnp.exp(-l)
            better = key < m
            m = jnp.where(better, key, m)
            idx = jnp.where(better, colv, idx)
            return m, idx

        return lax.fori_loop(0, NVEC, vec_body, (m, idx))

    m0 = jnp.full((16,), jnp.float32(float("inf")))
    i0 = jnp.zeros((16,), jnp.int32)
    m, idx = lax.fori_loop(0, NCH, chunk_body, (m0, i0))

    mbuf[...] = lax.bitcast_convert_type(m, jnp.int32)
    ibuf[...] = idx
    pltpu.sync_copy(mbuf, out_hbm.at[pl.ds(w * 32, 16)])
    pltpu.sync_copy(ibuf, out_hbm.at[pl.ds(w * 32 + 16, 16)])


def _sc_sample(logits_bot, interpret=False):
    return pl.kernel(
        _sc_body,
        out_type=jax.ShapeDtypeStruct((B_SC * 32,), jnp.int32),
        mesh=plsc.VectorSubcoreMesh(
            core_axis_name="c", subcore_axis_name="s", num_cores=2, num_subcores=16
        ),
        scratch_types=[
            pltpu.VMEM((CH,), jnp.float32),
            pltpu.VMEM((16,), jnp.int32),
            pltpu.VMEM((16,), jnp.int32),
        ],
        interpret=interpret,
    )(logits_bot)


@functools.partial(jax.jit, static_argnames=("interpret",))
def _sample(logits, interpret=False):
    out_sc = _sc_sample(logits[B_TC:].reshape(-1), interpret)
    out_tc = _tc_sample(logits[:B_TC], interpret)
    parts = out_sc.reshape(B_SC, 2, 16)
    lane = jnp.argmin(parts[:, 0, :], axis=1)
    best = jnp.take_along_axis(parts[:, 1, :], lane[:, None], axis=1)[:, 0]
    return jnp.concatenate([out_tc, best])


def kernel(logits):
    return _sample(logits).astype(jnp.int64)
